# baseline (speedup 1.0000x reference)
import jax
import jax.numpy as jnp
import numpy as np
from jax.experimental import pallas as pl
from jax.experimental.pallas import tpu as pltpu
from jax.experimental.pallas import tpu_sc as plsc

_LOG2 = float(np.log(2.0))

_GATHER_WINDOW = 256
_BEAD_BLOCK = 200
_N_CHUNKS = 10


def _sc_gather(features2d, idx2, start, rows):
    n_rows, d = features2d.shape
    w = _GATHER_WINDOW
    steps = rows // w
    step0 = start // w
    mesh = plsc.VectorSubcoreMesh(core_axis_name="core",
                                  subcore_axis_name="subcore")

    @pl.kernel(out_type=jax.ShapeDtypeStruct((rows, d), features2d.dtype),
               mesh=mesh)
    def gather_kernel(x_hbm, i_hbm, o_hbm):
        def body(i_vmem, o_vmem):
            pltpu.sync_copy(x_hbm.at[i_vmem.at[0]], o_vmem)

        pltpu.emit_pipeline(
            body,
            grid=(steps,),
            in_specs=[pl.BlockSpec((1, w),
                                   index_map=lambda i: (0, step0 + i))],
            out_specs=[pl.BlockSpec((w, d),
                                    index_map=lambda i: (i, 0))],
            core_axis_name=("core", "subcore"),
            dimension_semantics=(pltpu.PARALLEL,),
        )(i_hbm, o_hbm)

    return gather_kernel(features2d, idx2)


def _tc_body(rbf_ref, nf_ref, w1_ref, b1_ref, w2_ref, b2_ref, o_ref):
    b = o_ref.shape[0]
    x = rbf_ref[...]
    h = jnp.dot(x, w1_ref[...], preferred_element_type=jnp.float32) + b1_ref[...]
    h = jnp.maximum(h, 0.0) + jnp.log(1.0 + jnp.exp(-jnp.abs(h))) - _LOG2
    f = jnp.dot(h, w2_ref[...], preferred_element_type=jnp.float32) + b2_ref[...]
    prod = f * nf_ref[...]
    prod3 = prod.reshape(b, prod.shape[0] // b, prod.shape[1])
    o_ref[...] = prod3.sum(axis=1)


def _tc_filter_conv(rbf2d, nf, W1, b1, W2, b2, bead0, beads, n_neighbors):
    d = rbf2d.shape[1]
    k = W2.shape[1]
    B = _BEAD_BLOCK
    R = B * n_neighbors
    blk0 = bead0 // B
    return pl.pallas_call(
        _tc_body,
        grid=(beads // B,),
        in_specs=[
            pl.BlockSpec((R, d), lambda i: (blk0 + i, 0)),
            pl.BlockSpec((R, k), lambda i: (i, 0)),
            pl.BlockSpec((d, W1.shape[1]), lambda i: (0, 0)),
            pl.BlockSpec((1, W1.shape[1]), lambda i: (0, 0)),
            pl.BlockSpec((W2.shape[0], k), lambda i: (0, 0)),
            pl.BlockSpec((1, k), lambda i: (0, 0)),
        ],
        out_specs=pl.BlockSpec((B, k), lambda i: (i, 0)),
        out_shape=jax.ShapeDtypeStruct((beads, k), jnp.float32),
    )(rbf2d, nf, W1, b1, W2, b2)


def kernel(features, rbf_expansion, neighbor_list, neighbor_mask, W1, b1, W2, b2):
    n_frames, n_beads, n_filters = features.shape
    n_neighbors = neighbor_list.shape[2]
    n_gaussians = rbf_expansion.shape[3]

    feat2d = jnp.concatenate(
        [features.reshape(n_beads, n_filters),
         jnp.zeros((1, n_filters), jnp.float32)], axis=0)
    idx_flat = neighbor_list.reshape(1, n_beads * n_neighbors).astype(jnp.int32)
    mask_flat = neighbor_mask.reshape(1, n_beads * n_neighbors)
    idx2 = jnp.where(mask_flat > 0, idx_flat, jnp.int32(n_beads))

    rbf2d = rbf_expansion.reshape(n_beads * n_neighbors, n_gaussians)
    b1r = b1.reshape(1, n_filters)
    b2r = b2.reshape(1, n_filters)

    n_chunks = _N_CHUNKS
    cb = n_beads // n_chunks
    cr = cb * n_neighbors
    outs = []
    for c in range(n_chunks):
        nf_c = _sc_gather(feat2d, idx2, c * cr, cr)
        out_c = _tc_filter_conv(rbf2d, nf_c, W1, b1r, W2, b2r,
                                c * cb, cb, n_neighbors)
        outs.append(out_c)
    out = jnp.concatenate(outs, axis=0)
    return out.reshape(n_frames, n_beads, n_filters)


# speedup vs baseline: 1.0382x; 1.0382x over previous
import jax
import jax.numpy as jnp
import numpy as np
from jax.experimental import pallas as pl
from jax.experimental.pallas import tpu as pltpu
from jax.experimental.pallas import tpu_sc as plsc

_LOG2 = float(np.log(2.0))

_GATHER_WINDOW = 256
_BEAD_BLOCK = 400
_N_CHUNKS = 5


def _sc_gather(features2d, idx2, start, rows):
    n_rows, d = features2d.shape
    w = _GATHER_WINDOW
    steps = rows // w
    step0 = start // w
    mesh = plsc.VectorSubcoreMesh(core_axis_name="core",
                                  subcore_axis_name="subcore")

    @pl.kernel(out_type=jax.ShapeDtypeStruct((rows, d), features2d.dtype),
               mesh=mesh)
    def gather_kernel(x_hbm, i_hbm, o_hbm):
        def body(i_vmem, o_vmem):
            pltpu.sync_copy(x_hbm.at[i_vmem.at[0]], o_vmem)

        pltpu.emit_pipeline(
            body,
            grid=(steps,),
            in_specs=[pl.BlockSpec((1, w),
                                   index_map=lambda i: (0, step0 + i))],
            out_specs=[pl.BlockSpec((w, d),
                                    index_map=lambda i: (i, 0))],
            core_axis_name=("core", "subcore"),
            dimension_semantics=(pltpu.PARALLEL,),
        )(i_hbm, o_hbm)

    return gather_kernel(features2d, idx2)


def _tc_body(rbf_a, rbf_b, nf_a, nf_b, w1_ref, b1_ref, w2_ref, b2_ref, o_ref):
    b = o_ref.shape[0] // 2

    def half(rbf_ref, nf_ref):
        x = rbf_ref[...]
        h = (jnp.dot(x, w1_ref[...], preferred_element_type=jnp.float32)
             + b1_ref[...])
        h = jnp.maximum(h, 0.0) + jnp.log(1.0 + jnp.exp(-jnp.abs(h))) - _LOG2
        f = (jnp.dot(h, w2_ref[...], preferred_element_type=jnp.float32)
             + b2_ref[...])
        prod = f * nf_ref[...]
        prod3 = prod.reshape(b, prod.shape[0] // b, prod.shape[1])
        return prod3.sum(axis=1)

    o_ref[:b, :] = half(rbf_a, nf_a)
    o_ref[b:, :] = half(rbf_b, nf_b)


def _tc_filter_conv(rbf2d, nf, W1, b1, W2, b2, bead0, beads, n_neighbors):
    d = rbf2d.shape[1]
    k = W2.shape[1]
    B = _BEAD_BLOCK
    R = (B // 2) * n_neighbors
    blk0 = bead0 * 2 // B
    return pl.pallas_call(
        _tc_body,
        grid=(beads // B,),
        in_specs=[
            pl.BlockSpec((R, d), lambda i: (blk0 + 2 * i, 0)),
            pl.BlockSpec((R, d), lambda i: (blk0 + 2 * i + 1, 0)),
            pl.BlockSpec((R, k), lambda i: (2 * i, 0)),
            pl.BlockSpec((R, k), lambda i: (2 * i + 1, 0)),
            pl.BlockSpec((d, W1.shape[1]), lambda i: (0, 0)),
            pl.BlockSpec((1, W1.shape[1]), lambda i: (0, 0)),
            pl.BlockSpec((W2.shape[0], k), lambda i: (0, 0)),
            pl.BlockSpec((1, k), lambda i: (0, 0)),
        ],
        out_specs=pl.BlockSpec((B, k), lambda i: (i, 0)),
        out_shape=jax.ShapeDtypeStruct((beads, k), jnp.float32),
    )(rbf2d, rbf2d, nf, nf, W1, b1, W2, b2)


def kernel(features, rbf_expansion, neighbor_list, neighbor_mask, W1, b1, W2, b2):
    n_frames, n_beads, n_filters = features.shape
    n_neighbors = neighbor_list.shape[2]
    n_gaussians = rbf_expansion.shape[3]

    feat2d = jnp.concatenate(
        [features.reshape(n_beads, n_filters),
         jnp.zeros((1, n_filters), jnp.float32)], axis=0)
    idx_flat = neighbor_list.reshape(1, n_beads * n_neighbors).astype(jnp.int32)
    mask_flat = neighbor_mask.reshape(1, n_beads * n_neighbors)
    idx2 = jnp.where(mask_flat > 0, idx_flat, jnp.int32(n_beads))

    rbf2d = rbf_expansion.reshape(n_beads * n_neighbors, n_gaussians)
    b1r = b1.reshape(1, n_filters)
    b2r = b2.reshape(1, n_filters)

    n_chunks = _N_CHUNKS
    cb = n_beads // n_chunks
    cr = cb * n_neighbors
    outs = []
    for c in range(n_chunks):
        nf_c = _sc_gather(feat2d, idx2, c * cr, cr)
        out_c = _tc_filter_conv(rbf2d, nf_c, W1, b1r, W2, b2r,
                                c * cb, cb, n_neighbors)
        outs.append(out_c)
    out = jnp.concatenate(outs, axis=0)
    return out.reshape(n_frames, n_beads, n_filters)


# speedup vs baseline: 1.0422x; 1.0038x over previous
import jax
import jax.numpy as jnp
import numpy as np
from jax.experimental import pallas as pl
from jax.experimental.pallas import tpu as pltpu
from jax.experimental.pallas import tpu_sc as plsc

_LOG2 = float(np.log(2.0))

_GATHER_WINDOW = 256
_BEAD_BLOCK = 400
_N_CHUNKS = 5


def _sc_gather(features2d, idx2, start, rows):
    n_rows, d = features2d.shape
    w = _GATHER_WINDOW
    steps = rows // w
    step0 = start // w
    mesh = plsc.VectorSubcoreMesh(core_axis_name="core",
                                  subcore_axis_name="subcore")

    @pl.kernel(out_type=jax.ShapeDtypeStruct((rows, d), features2d.dtype),
               mesh=mesh)
    def gather_kernel(x_hbm, i_hbm, o_hbm):
        def body(i_vmem, o_vmem):
            pltpu.sync_copy(x_hbm.at[i_vmem.at[0]], o_vmem)

        pltpu.emit_pipeline(
            body,
            grid=(steps,),
            in_specs=[pl.BlockSpec((1, w),
                                   index_map=lambda i: (0, step0 + i))],
            out_specs=[pl.BlockSpec((w, d),
                                    index_map=lambda i: (i, 0))],
            core_axis_name=("core", "subcore"),
            dimension_semantics=(pltpu.PARALLEL,),
        )(i_hbm, o_hbm)

    return gather_kernel(features2d, idx2)


def _tc_body(rbf_ref, nf_ref, w1_ref, b1_ref, w2_ref, b2_ref, o_ref):
    b = o_ref.shape[0]
    x = rbf_ref[...]
    h = jnp.dot(x, w1_ref[...], preferred_element_type=jnp.float32) + b1_ref[...]
    h = jnp.maximum(h, 0.0) + jnp.log(1.0 + jnp.exp(-jnp.abs(h))) - _LOG2
    f = jnp.dot(h, w2_ref[...], preferred_element_type=jnp.float32) + b2_ref[...]
    prod = f * nf_ref[...]
    prod3 = prod.reshape(b, prod.shape[0] // b, prod.shape[1])
    o_ref[...] = prod3.sum(axis=1)


def _tc_filter_conv(rbf2d, nf, W1, b1, W2, b2, bead0, beads, n_neighbors):
    d = rbf2d.shape[1]
    k = W2.shape[1]
    B = _BEAD_BLOCK
    R = B * n_neighbors
    blk0 = bead0 // B
    return pl.pallas_call(
        _tc_body,
        grid=(beads // B,),
        in_specs=[
            pl.BlockSpec((R, d), lambda i: (blk0 + i, 0)),
            pl.BlockSpec((R, k), lambda i: (i, 0)),
            pl.BlockSpec((d, W1.shape[1]), lambda i: (0, 0)),
            pl.BlockSpec((1, W1.shape[1]), lambda i: (0, 0)),
            pl.BlockSpec((W2.shape[0], k), lambda i: (0, 0)),
            pl.BlockSpec((1, k), lambda i: (0, 0)),
        ],
        out_specs=pl.BlockSpec((B, k), lambda i: (i, 0)),
        out_shape=jax.ShapeDtypeStruct((beads, k), jnp.float32),
    )(rbf2d, nf, W1, b1, W2, b2)


def kernel(features, rbf_expansion, neighbor_list, neighbor_mask, W1, b1, W2, b2):
    n_frames, n_beads, n_filters = features.shape
    n_neighbors = neighbor_list.shape[2]
    n_gaussians = rbf_expansion.shape[3]

    feat2d = jnp.concatenate(
        [features.reshape(n_beads, n_filters),
         jnp.zeros((1, n_filters), jnp.float32)], axis=0)
    idx_flat = neighbor_list.reshape(1, n_beads * n_neighbors).astype(jnp.int32)
    mask_flat = neighbor_mask.reshape(1, n_beads * n_neighbors)
    idx2 = jnp.where(mask_flat > 0, idx_flat, jnp.int32(n_beads))

    rbf2d = rbf_expansion.reshape(n_beads * n_neighbors, n_gaussians)
    b1r = b1.reshape(1, n_filters)
    b2r = b2.reshape(1, n_filters)

    n_chunks = _N_CHUNKS
    cb = n_beads // n_chunks
    cr = cb * n_neighbors
    outs = []
    for c in range(n_chunks):
        nf_c = _sc_gather(feat2d, idx2, c * cr, cr)
        out_c = _tc_filter_conv(rbf2d, nf_c, W1, b1r, W2, b2r,
                                c * cb, cb, n_neighbors)
        outs.append(out_c)
    out = jnp.concatenate(outs, axis=0)
    return out.reshape(n_frames, n_beads, n_filters)
